# baseline (speedup 1.0000x reference)
import functools

import jax
import jax.numpy as jnp
from jax.experimental import pallas as pl
from jax.experimental.pallas import tpu as pltpu
from jax.experimental.pallas import tpu_sc as plsc

K = 32
_PREP_RB = 1000
_TOP_RB = 200
_AGG_RB = 1000
_GATHER_WIN = 80


def _dot(a, b):
    return jax.lax.dot_general(
        a, b, dimension_numbers=(((1,), (0,)), ((), ())),
        preferred_element_type=jnp.float32,
        precision=jax.lax.Precision.DEFAULT)


def _dot_t(a, b):
    return jax.lax.dot_general(
        a, b, dimension_numbers=(((1,), (1,)), ((), ())),
        preferred_element_type=jnp.float32,
        precision=jax.lax.Precision.DEFAULT)


def _prep_body(x_ref, w1_ref, b1_ref, w2_ref, b2_ref, wa_ref, bm1_ref, wb_ref,
               mapped_ref, a_ref, b_ref):
    xb = x_ref[...]
    h = jnp.maximum(_dot(xb, w1_ref[...]) + b1_ref[...], 0.0)
    mapped_ref[...] = _dot(h, w2_ref[...]) + b2_ref[...]
    a_ref[...] = _dot(xb, wa_ref[...]) + bm1_ref[...]
    b_ref[...] = _dot(xb, wb_ref[...])


def _prep(x, W1, b1, W2, b2, Wa, bm1, Wb):
    n, d = x.shape
    rb = _PREP_RB
    full = lambda s: pl.BlockSpec(s, lambda i: (0,) * len(s))
    row = pl.BlockSpec((rb, d), lambda i: (i, 0))
    out_sds = jax.ShapeDtypeStruct((n, d), jnp.float32)
    return pl.pallas_call(
        _prep_body,
        grid=(n // rb,),
        in_specs=[row, full((d, d)), full((1, d)), full((d, d)), full((1, d)),
                  full((d, d)), full((1, d)), full((d, d))],
        out_specs=[row, row, row],
        out_shape=[out_sds, out_sds, out_sds],
    )(x, W1, b1.reshape(1, d), W2, b2.reshape(1, d), Wa, bm1.reshape(1, d), Wb)


_G = 128


def _simgroups_body(mb_ref, mf_ref, s3_ref, grp_ref, *, n_real):
    npad = mf_ref.shape[0]
    ng = npad // _G
    rb = mb_ref.shape[0]
    i = pl.program_id(0)
    s = _dot_t(mb_ref[...], mf_ref[...])
    rows = jax.lax.broadcasted_iota(jnp.int32, (rb, npad), 0) + i * rb
    cols = jax.lax.broadcasted_iota(jnp.int32, (rb, npad), 1)
    s = jnp.where(cols == rows, -1e9, s)
    s = jnp.where(cols >= n_real, -1e30, s)
    gms = []
    for j in range(ng):
        chunk = s[:, _G * j:_G * (j + 1)]
        s3_ref[:, j, :] = chunk
        gms.append(jnp.max(chunk, axis=1, keepdims=True))
    gm0 = jnp.concatenate(gms, axis=1)
    lane_g = jax.lax.broadcasted_iota(jnp.int32, (rb, ng), 1)
    lane_k = jax.lax.broadcasted_iota(jnp.int32, (rb, K), 1)

    def pass_k(k, carry):
        gm, grp = carry
        m = jnp.max(gm, axis=1, keepdims=True)
        e = gm == m
        g = jnp.max(jnp.where(e, lane_g, -1), axis=1, keepdims=True)
        gm = jnp.where(e, -3e38, gm)
        grp = jnp.where(lane_k == k, g, grp)
        return gm, grp

    grp0 = jnp.zeros((rb, K), jnp.int32)
    _, grp = jax.lax.fori_loop(0, K, pass_k, (gm0, grp0))
    row_g = jax.lax.broadcasted_iota(jnp.int32, (rb, K), 0) + i * rb
    grp_ref[...] = row_g * ng + grp


def _simgroups(mapped_pad, n):
    npad, d = mapped_pad.shape
    ng = npad // _G
    rb = _TOP_RB
    return pl.pallas_call(
        functools.partial(_simgroups_body, n_real=n),
        grid=(n // rb,),
        in_specs=[pl.BlockSpec((rb, d), lambda i: (i, 0)),
                  pl.BlockSpec((npad, d), lambda i: (0, 0))],
        out_specs=[pl.BlockSpec((rb, ng, _G), lambda i: (i, 0, 0)),
                   pl.BlockSpec((rb, K), lambda i: (i, 0))],
        out_shape=[jax.ShapeDtypeStruct((n, ng, _G), jnp.float32),
                   jax.ShapeDtypeStruct((n, K), jnp.int32)],
    )(mapped_pad, mapped_pad)


def _select_body(c_ref, grp_ref, w_ref, nbr_ref, cs_ref, *, ng):
    rb, w_ = c_ref.shape
    i = pl.program_id(0)
    cs_ref[...] = c_ref[...]
    row_g = jax.lax.broadcasted_iota(jnp.int32, (rb, K), 0) + i * rb
    glocal = grp_ref[...] - row_g * ng
    flat = jax.lax.broadcasted_iota(jnp.int32, (rb, w_), 1)
    lane_k = jax.lax.broadcasted_iota(jnp.int32, (rb, K), 1)

    def pass_k(k, carry):
        v, idx = carry
        c = cs_ref[...]
        m = jnp.max(c, axis=1, keepdims=True)
        e = c == m
        f = jnp.max(jnp.where(e, flat, -1), axis=1, keepdims=True)
        cs_ref[...] = jnp.where(e, -3e38, c)
        v = jnp.where(lane_k == k, m, v)
        idx = jnp.where(lane_k == k, f, idx)
        return v, idx

    v0 = jnp.full((rb, K), -3e38, jnp.float32)
    i0 = jnp.zeros((rb, K), jnp.int32)
    v, fidx = jax.lax.fori_loop(0, K, pass_k, (v0, i0))
    k2 = fidx // _G
    t = fidx - k2 * _G
    gsel = jnp.zeros((rb, K), jnp.int32)
    for k2i in range(K):
        gsel = jnp.where(k2 == k2i, glocal[:, k2i:k2i + 1], gsel)
    mx = jnp.max(v, axis=1, keepdims=True)
    ex = jnp.exp(v - mx)
    w_ref[...] = ex / jnp.sum(ex, axis=1, keepdims=True)
    nbr_ref[...] = gsel * _G + t


def _select(c, grp_g, ng):
    n, w_ = c.shape
    rb = _TOP_RB
    return pl.pallas_call(
        functools.partial(_select_body, ng=ng),
        grid=(n // rb,),
        in_specs=[pl.BlockSpec((rb, w_), lambda i: (i, 0)),
                  pl.BlockSpec((rb, K), lambda i: (i, 0))],
        out_specs=[pl.BlockSpec((rb, K), lambda i: (i, 0)),
                   pl.BlockSpec((rb, K), lambda i: (i, 0))],
        out_shape=[jax.ShapeDtypeStruct((n, K), jnp.float32),
                   jax.ShapeDtypeStruct((n, K), jnp.int32)],
        scratch_shapes=[pltpu.VMEM((rb, w_), jnp.float32)],
    )(c, grp_g)


def _sc_gather(b, idx_flat):
    e = idx_flat.shape[0]
    d = b.shape[1]
    win = _GATHER_WIN
    nc, ns = 2, 16
    nw = nc * ns
    per_w = e // nw
    mesh = plsc.VectorSubcoreMesh(core_axis_name="c", subcore_axis_name="s")

    @functools.partial(
        pl.kernel,
        out_type=jax.ShapeDtypeStruct((e, d), b.dtype),
        mesh=mesh,
        scratch_types=[pltpu.VMEM((per_w,), jnp.int32),
                       pltpu.VMEM((win, d), b.dtype),
                       pltpu.VMEM((win, d), b.dtype),
                       pltpu.SemaphoreType.DMA,
                       pltpu.SemaphoreType.DMA,
                       pltpu.SemaphoreType.DMA,
                       pltpu.SemaphoreType.DMA],
    )
    def gk(b_hbm, i_hbm, o_hbm, idx_v, rows_a, rows_b, ga, gb, oa, ob):
        wid = jax.lax.axis_index("s") * nc + jax.lax.axis_index("c")
        base = wid * per_w
        pltpu.sync_copy(i_hbm.at[pl.ds(base, per_w)], idx_v)

        def gather(off, buf, sem):
            return pltpu.async_copy(
                b_hbm.at[idx_v.at[pl.ds(off, win)]], buf, sem)

        def put(off, buf, sem):
            return pltpu.async_copy(buf, o_hbm.at[pl.ds(base + off, win)], sem)

        npair = (per_w // win) // 2

        @pl.loop(0, npair)
        def _(p):
            off = 2 * win * p
            ca = gather(off, rows_a, ga)
            cb = gather(off + win, rows_b, gb)
            ca.wait()
            pa = put(off, rows_a, oa)
            cb.wait()
            pb = put(off + win, rows_b, ob)
            pa.wait()
            pb.wait()

        if (per_w // win) % 2:
            off = 2 * win * npair
            gather(off, rows_a, ga).wait()
            put(off, rows_a, oa).wait()

    return gk(b, idx_flat)


def _agg_body(g_ref, a_ref, w_ref, wm2_ref, bm2_ref, o_ref):
    rb, _, d = g_ref.shape
    a = a_ref[...]
    w = w_ref[...]
    acc = jnp.zeros((rb, d), jnp.float32)
    for k in range(K):
        wk = jnp.broadcast_to(w[:, k:k + 1], (rb, d))
        acc = acc + jnp.maximum(a + g_ref[:, k, :], 0.0) * wk
    o_ref[...] = (_dot(acc, wm2_ref[...]) + bm2_ref[...]) * (1.0 / K)


def _agg(g, a, w, Wm2, bm2):
    n, d = a.shape
    rb = _AGG_RB
    return pl.pallas_call(
        _agg_body,
        grid=(n // rb,),
        in_specs=[pl.BlockSpec((rb, K, d), lambda i: (i, 0, 0)),
                  pl.BlockSpec((rb, d), lambda i: (i, 0)),
                  pl.BlockSpec((rb, K), lambda i: (i, 0)),
                  pl.BlockSpec((d, d), lambda i: (0, 0)),
                  pl.BlockSpec((1, d), lambda i: (0, 0))],
        out_specs=pl.BlockSpec((rb, d), lambda i: (i, 0)),
        out_shape=jax.ShapeDtypeStruct((n, d), jnp.float32),
    )(g, a, w, Wm2, bm2.reshape(1, d))


def kernel(x, W1, b1, W2, b2, Wm1, bm1, Wm2, bm2):
    n, d = x.shape
    Wa = Wm1[:d] - Wm1[d:]
    Wb = Wm1[d:]
    mapped, a, bfeat = _prep(x, W1, b1, W2, b2, Wa, bm1, Wb)
    npad = ((n + _G - 1) // _G) * _G
    mapped_pad = jnp.concatenate(
        [mapped, jnp.zeros((npad - n, d), jnp.float32)], axis=0)
    ng = npad // _G
    s3, grp_g = _simgroups(mapped_pad, n)
    cand = _sc_gather(s3.reshape(n * ng, _G), grp_g.reshape(n * K))
    w, nbr = _select(cand.reshape(n, K * _G), grp_g, ng)
    g = _sc_gather(bfeat, nbr.reshape(n * K))
    out = _agg(g.reshape(n, K, d), a, w, Wm2, bm2)
    return out


# speedup vs baseline: 1.0354x; 1.0354x over previous
import functools

import jax
import jax.numpy as jnp
from jax.experimental import pallas as pl
from jax.experimental.pallas import tpu as pltpu
from jax.experimental.pallas import tpu_sc as plsc

K = 32
_PREP_RB = 1000
_TOP_RB = 200
_AGG_RB = 1000
_GATHER_WIN = 80


def _dot(a, b):
    return jax.lax.dot_general(
        a, b, dimension_numbers=(((1,), (0,)), ((), ())),
        preferred_element_type=jnp.float32,
        precision=jax.lax.Precision.DEFAULT)


def _dot_t(a, b):
    return jax.lax.dot_general(
        a, b, dimension_numbers=(((1,), (1,)), ((), ())),
        preferred_element_type=jnp.float32,
        precision=jax.lax.Precision.DEFAULT)


def _prep_body(x_ref, w1_ref, b1_ref, w2_ref, b2_ref, wa_ref, bm1_ref, wb_ref,
               mapped_ref, a_ref, b_ref):
    xb = x_ref[...]
    h = jnp.maximum(_dot(xb, w1_ref[...]) + b1_ref[...], 0.0)
    mapped_ref[...] = _dot(h, w2_ref[...]) + b2_ref[...]
    a_ref[...] = _dot(xb, wa_ref[...]) + bm1_ref[...]
    b_ref[...] = _dot(xb, wb_ref[...])


def _prep(x, W1, b1, W2, b2, Wa, bm1, Wb):
    n, d = x.shape
    rb = _PREP_RB
    full = lambda s: pl.BlockSpec(s, lambda i: (0,) * len(s))
    row = pl.BlockSpec((rb, d), lambda i: (i, 0))
    out_sds = jax.ShapeDtypeStruct((n, d), jnp.float32)
    return pl.pallas_call(
        _prep_body,
        grid=(n // rb,),
        in_specs=[row, full((d, d)), full((1, d)), full((d, d)), full((1, d)),
                  full((d, d)), full((1, d)), full((d, d))],
        out_specs=[row, row, row],
        out_shape=[out_sds, out_sds, out_sds],
    )(x, W1, b1.reshape(1, d), W2, b2.reshape(1, d), Wa, bm1.reshape(1, d), Wb)


_G = 128


def _simgroups_body(mb_ref, mf_ref, s3_ref, grp_ref, *, n_real):
    npad = mf_ref.shape[0]
    ng = npad // _G
    rb = mb_ref.shape[0]
    i = pl.program_id(0)
    s = _dot_t(mb_ref[...], mf_ref[...])
    rows = jax.lax.broadcasted_iota(jnp.int32, (rb, npad), 0) + i * rb
    cols = jax.lax.broadcasted_iota(jnp.int32, (rb, npad), 1)
    s = jnp.where(cols == rows, -1e9, s)
    s = jnp.where(cols >= n_real, -1e30, s)
    gms = []
    for j in range(ng):
        chunk = s[:, _G * j:_G * (j + 1)]
        s3_ref[:, j, :] = chunk
        gms.append(jnp.max(chunk, axis=1, keepdims=True))
    gm0 = jnp.concatenate(gms, axis=1)
    lane_g = jax.lax.broadcasted_iota(jnp.int32, (rb, ng), 1)
    lane_k = jax.lax.broadcasted_iota(jnp.int32, (rb, K), 1)

    def pass_k(k, carry):
        gm, grp = carry
        m = jnp.max(gm, axis=1, keepdims=True)
        e = gm == m
        g = jnp.max(jnp.where(e, lane_g, -1), axis=1, keepdims=True)
        gm = jnp.where(e, -3e38, gm)
        grp = jnp.where(lane_k == k, g, grp)
        return gm, grp

    grp0 = jnp.zeros((rb, K), jnp.int32)
    _, grp = jax.lax.fori_loop(0, K, pass_k, (gm0, grp0))
    row_g = jax.lax.broadcasted_iota(jnp.int32, (rb, K), 0) + i * rb
    grp_ref[...] = row_g * ng + grp


def _simgroups(mapped_pad, n):
    npad, d = mapped_pad.shape
    ng = npad // _G
    rb = _TOP_RB
    return pl.pallas_call(
        functools.partial(_simgroups_body, n_real=n),
        grid=(n // rb,),
        in_specs=[pl.BlockSpec((rb, d), lambda i: (i, 0)),
                  pl.BlockSpec((npad, d), lambda i: (0, 0))],
        out_specs=[pl.BlockSpec((rb, ng, _G), lambda i: (i, 0, 0)),
                   pl.BlockSpec((rb, K), lambda i: (i, 0))],
        out_shape=[jax.ShapeDtypeStruct((n, ng, _G), jnp.float32),
                   jax.ShapeDtypeStruct((n, K), jnp.int32)],
    )(mapped_pad, mapped_pad)


def _select_body(c_ref, grp_ref, w_ref, nbr_ref, cs_ref, *, ng):
    rb, w_ = c_ref.shape
    i = pl.program_id(0)
    cs_ref[...] = c_ref[...]
    row_g = jax.lax.broadcasted_iota(jnp.int32, (rb, K), 0) + i * rb
    glocal = grp_ref[...] - row_g * ng
    flat = jax.lax.broadcasted_iota(jnp.int32, (rb, w_), 1)
    lane_k = jax.lax.broadcasted_iota(jnp.int32, (rb, K), 1)

    def pass_k(k, carry):
        v, idx = carry
        c = cs_ref[...]
        m = jnp.max(c, axis=1, keepdims=True)
        e = c == m
        f = jnp.max(jnp.where(e, flat, -1), axis=1, keepdims=True)
        cs_ref[...] = jnp.where(e, -3e38, c)
        v = jnp.where(lane_k == k, m, v)
        idx = jnp.where(lane_k == k, f, idx)
        return v, idx

    v0 = jnp.full((rb, K), -3e38, jnp.float32)
    i0 = jnp.zeros((rb, K), jnp.int32)
    v, fidx = jax.lax.fori_loop(0, K, pass_k, (v0, i0))
    k2 = fidx // _G
    t = fidx - k2 * _G
    gsel = jnp.zeros((rb, K), jnp.int32)
    for k2i in range(K):
        gsel = jnp.where(k2 == k2i, glocal[:, k2i:k2i + 1], gsel)
    mx = jnp.max(v, axis=1, keepdims=True)
    ex = jnp.exp(v - mx)
    w_ref[...] = ex / jnp.sum(ex, axis=1, keepdims=True)
    nbr_ref[...] = gsel * _G + t


def _select(c, grp_g, ng):
    n, w_ = c.shape
    rb = _TOP_RB
    return pl.pallas_call(
        functools.partial(_select_body, ng=ng),
        grid=(n // rb,),
        in_specs=[pl.BlockSpec((rb, w_), lambda i: (i, 0)),
                  pl.BlockSpec((rb, K), lambda i: (i, 0))],
        out_specs=[pl.BlockSpec((rb, K), lambda i: (i, 0)),
                   pl.BlockSpec((rb, K), lambda i: (i, 0))],
        out_shape=[jax.ShapeDtypeStruct((n, K), jnp.float32),
                   jax.ShapeDtypeStruct((n, K), jnp.int32)],
        scratch_shapes=[pltpu.VMEM((rb, w_), jnp.float32)],
    )(c, grp_g)


def _sc_gather(b, idx_flat):
    e = idx_flat.shape[0]
    d = b.shape[1]
    win = _GATHER_WIN
    nc, ns = 2, 16
    nw = nc * ns
    per_w = e // nw
    mesh = plsc.VectorSubcoreMesh(core_axis_name="c", subcore_axis_name="s")

    @functools.partial(
        pl.kernel,
        out_type=jax.ShapeDtypeStruct((e, d), b.dtype),
        mesh=mesh,
        scratch_types=[pltpu.VMEM((per_w,), jnp.int32),
                       pltpu.VMEM((win, d), b.dtype),
                       pltpu.VMEM((win, d), b.dtype),
                       pltpu.SemaphoreType.DMA,
                       pltpu.SemaphoreType.DMA,
                       pltpu.SemaphoreType.DMA,
                       pltpu.SemaphoreType.DMA],
    )
    def gk(b_hbm, i_hbm, o_hbm, idx_v, rows_a, rows_b, ga, gb, oa, ob):
        wid = jax.lax.axis_index("s") * nc + jax.lax.axis_index("c")
        base = wid * per_w
        pltpu.sync_copy(i_hbm.at[pl.ds(base, per_w)], idx_v)

        def gather(off, buf, sem):
            return pltpu.async_copy(
                b_hbm.at[idx_v.at[pl.ds(off, win)]], buf, sem)

        def put(off, buf, sem):
            return pltpu.async_copy(buf, o_hbm.at[pl.ds(base + off, win)], sem)

        npair = (per_w // win) // 2

        @pl.loop(0, npair)
        def _(p):
            off = 2 * win * p
            ca = gather(off, rows_a, ga)
            cb = gather(off + win, rows_b, gb)
            ca.wait()
            pa = put(off, rows_a, oa)
            cb.wait()
            pb = put(off + win, rows_b, ob)
            pa.wait()
            pb.wait()

        if (per_w // win) % 2:
            off = 2 * win * npair
            gather(off, rows_a, ga).wait()
            put(off, rows_a, oa).wait()

    return gk(b, idx_flat)


def _agg_body(g_ref, a_ref, w_ref, wm2_ref, bm2_ref, o_ref):
    h = jnp.maximum(a_ref[...][:, None, :] + g_ref[...], 0.0)
    agg = jnp.sum(h * w_ref[...][:, :, None], axis=1)
    o_ref[...] = (_dot(agg, wm2_ref[...]) + bm2_ref[...]) * (1.0 / K)


def _agg(g, a, w, Wm2, bm2):
    n, d = a.shape
    rb = _AGG_RB
    return pl.pallas_call(
        _agg_body,
        grid=(n // rb,),
        in_specs=[pl.BlockSpec((rb, K, d), lambda i: (i, 0, 0)),
                  pl.BlockSpec((rb, d), lambda i: (i, 0)),
                  pl.BlockSpec((rb, K), lambda i: (i, 0)),
                  pl.BlockSpec((d, d), lambda i: (0, 0)),
                  pl.BlockSpec((1, d), lambda i: (0, 0))],
        out_specs=pl.BlockSpec((rb, d), lambda i: (i, 0)),
        out_shape=jax.ShapeDtypeStruct((n, d), jnp.float32),
    )(g, a, w, Wm2, bm2.reshape(1, d))


def kernel(x, W1, b1, W2, b2, Wm1, bm1, Wm2, bm2):
    n, d = x.shape
    Wa = Wm1[:d] - Wm1[d:]
    Wb = Wm1[d:]
    mapped, a, bfeat = _prep(x, W1, b1, W2, b2, Wa, bm1, Wb)
    npad = ((n + _G - 1) // _G) * _G
    mapped_pad = jnp.concatenate(
        [mapped, jnp.zeros((npad - n, d), jnp.float32)], axis=0)
    ng = npad // _G
    s3, grp_g = _simgroups(mapped_pad, n)
    cand = _sc_gather(s3.reshape(n * ng, _G), grp_g.reshape(n * K))
    w, nbr = _select(cand.reshape(n, K * _G), grp_g, ng)
    g = _sc_gather(bfeat, nbr.reshape(n * K))
    out = _agg(g.reshape(n, K, d), a, w, Wm2, bm2)
    return out
